# baseline (speedup 1.0000x reference)
import functools

import jax
import jax.numpy as jnp
from jax import lax
from jax.experimental import pallas as pl
from jax.experimental.pallas import tpu as pltpu
from jax.experimental.pallas import tpu_sc as plsc

BATCH = 16384
EMBED_DIM = 32
NUM_CORES = 2
NUM_SUBCORES = 16
NW = NUM_CORES * NUM_SUBCORES
BPW = BATCH // NW
CHUNK = 128
KCH = BPW // CHUNK

MLP_BLK = 2048


def _make_sc_gather():
    mesh = plsc.VectorSubcoreMesh(core_axis_name="c", subcore_axis_name="s")

    @functools.partial(
        pl.kernel,
        mesh=mesh,
        compiler_params=pltpu.CompilerParams(use_tc_tiling_on_sc=False),
        out_type=[
            jax.ShapeDtypeStruct((BATCH, EMBED_DIM), jnp.float32),
            jax.ShapeDtypeStruct((BATCH, EMBED_DIM), jnp.float32),
        ],
        scratch_types=[
            pltpu.VMEM((KCH, CHUNK), jnp.int32),
            pltpu.VMEM((KCH, CHUNK), jnp.int32),
            pltpu.VMEM((BPW, EMBED_DIM), jnp.float32),
            pltpu.VMEM((BPW, EMBED_DIM), jnp.float32),
            pltpu.SemaphoreType.DMA,
        ],
    )
    def gather(uid_hbm, iid_hbm, uemb_hbm, iemb_hbm, ue_out, ie_out,
               uidx_v, iidx_v, urows_v, irows_v, sem):
        wid = lax.axis_index("s") * NUM_CORES + lax.axis_index("c")
        base = wid * BPW
        pltpu.sync_copy(uid_hbm.at[wid], uidx_v)
        pltpu.sync_copy(iid_hbm.at[wid], iidx_v)
        copies = []
        for j in range(KCH):
            copies.append(pltpu.async_copy(
                uemb_hbm.at[uidx_v.at[j]],
                urows_v.at[pl.ds(j * CHUNK, CHUNK)], sem))
            copies.append(pltpu.async_copy(
                iemb_hbm.at[iidx_v.at[j]],
                irows_v.at[pl.ds(j * CHUNK, CHUNK)], sem))
        for c in copies:
            c.wait()
        pltpu.sync_copy(urows_v, ue_out.at[pl.ds(base, BPW)])
        pltpu.sync_copy(irows_v, ie_out.at[pl.ds(base, BPW)])

    return gather


_sc_gather = _make_sc_gather()


def _mlp_body(ue, ie, w0u, w0i, b0, w1, b1, w2, b2, wo, bo, out):
    h = jnp.dot(ue[...], w0u[...], preferred_element_type=jnp.float32)
    h = h + jnp.dot(ie[...], w0i[...], preferred_element_type=jnp.float32)
    h = jnp.maximum(h + b0[...], 0.0)
    h = jnp.maximum(jnp.dot(h, w1[...], preferred_element_type=jnp.float32) + b1[...], 0.0)
    h = jnp.maximum(jnp.dot(h, w2[...], preferred_element_type=jnp.float32) + b2[...], 0.0)
    o = jnp.dot(h, wo[...], preferred_element_type=jnp.float32) + bo[...]
    out[...] = o[:, 0]


def _tc_mlp(ue, ie, W0u, W0i, b0, W1, b1, W2, b2, Wo, bo):
    grid = (BATCH // MLP_BLK,)
    full = lambda shape: pl.BlockSpec(shape, lambda i: (0,) * len(shape))
    return pl.pallas_call(
        _mlp_body,
        grid=grid,
        in_specs=[
            pl.BlockSpec((MLP_BLK, EMBED_DIM), lambda i: (i, 0)),
            pl.BlockSpec((MLP_BLK, EMBED_DIM), lambda i: (i, 0)),
            full(W0u.shape), full(W0i.shape), full(b0.shape),
            full(W1.shape), full(b1.shape),
            full(W2.shape), full(b2.shape),
            full(Wo.shape), full(bo.shape),
        ],
        out_specs=pl.BlockSpec((MLP_BLK,), lambda i: (i,)),
        out_shape=jax.ShapeDtypeStruct((BATCH,), jnp.float32),
    )(ue, ie, W0u, W0i, b0, W1, b1, W2, b2, Wo, bo)


def kernel(user_ids, item_ids, user_emb, item_emb, W0, b0, W1, b1, W2, b2, Wo, bo):
    uid = user_ids.reshape(NW, KCH, CHUNK)
    iid = item_ids.reshape(NW, KCH, CHUNK)
    ue, ie = _sc_gather(uid, iid, user_emb, item_emb)
    W0u = W0[:EMBED_DIM]
    W0i = W0[EMBED_DIM:]
    return _tc_mlp(
        ue, ie, W0u, W0i,
        b0.reshape(1, -1), W1, b1.reshape(1, -1),
        W2, b2.reshape(1, -1), Wo, bo.reshape(1, 1),
    )
